# baseline (speedup 1.0000x reference)
import functools
import jax
import jax.numpy as jnp
from jax import lax
from jax.experimental import pallas as pl
from jax.experimental.pallas import tpu as pltpu
from jax.experimental.pallas import tpu_sc as plsc

_CH = 32768


def kernel(x, pos):
    B, T, D = x.shape
    total = B * T * D
    pos_n = T * D

    info = plsc.get_sparse_core_info()
    nc, ns = info.num_cores, info.num_subcores
    nw = nc * ns
    per_w = total // nw
    assert total % nw == 0
    assert pos_n % per_w == 0
    w_per_batch = pos_n // per_w
    ch = min(_CH, per_w)
    niter = per_w // ch
    assert per_w % ch == 0

    xf = x.reshape(total)
    pf = pos.reshape(-1)[:pos_n]

    mesh = plsc.VectorSubcoreMesh(core_axis_name="c", subcore_axis_name="s")

    @functools.partial(
        pl.kernel,
        out_type=jax.ShapeDtypeStruct((total,), jnp.float32),
        mesh=mesh,
        scratch_types=[
            pltpu.VMEM((ch,), jnp.float32),
            pltpu.VMEM((ch,), jnp.float32),
        ],
    )
    def k(x_hbm, p_hbm, o_hbm, xb, pb):
        wid = lax.axis_index("c") * ns + lax.axis_index("s")
        base = wid * per_w
        pbase = (wid % w_per_batch) * per_w

        @pl.loop(0, niter)
        def _(i):
            off = base + i * ch
            poff = pbase + i * ch
            pltpu.sync_copy(x_hbm.at[pl.ds(off, ch)], xb)
            pltpu.sync_copy(p_hbm.at[pl.ds(poff, ch)], pb)

            @pl.loop(0, ch, step=16, unroll=8)
            def _(j):
                xb[pl.ds(j, 16)] = xb[pl.ds(j, 16)] + pb[pl.ds(j, 16)]

            pltpu.sync_copy(xb, o_hbm.at[pl.ds(off, ch)])

    return k(xf, pf).reshape(B, T, D)


# speedup vs baseline: 1.1852x; 1.1852x over previous
import functools
import jax
import jax.numpy as jnp
from jax import lax
from jax.experimental import pallas as pl
from jax.experimental.pallas import tpu as pltpu
from jax.experimental.pallas import tpu_sc as plsc

_CH = 16384


def kernel(x, pos):
    B, T, D = x.shape
    total = B * T * D
    pos_n = T * D

    info = plsc.get_sparse_core_info()
    nc, ns = info.num_cores, info.num_subcores
    nw = nc * ns
    per_w = total // nw
    assert total % nw == 0
    assert pos_n % per_w == 0
    w_per_batch = pos_n // per_w
    ch = min(_CH, per_w)
    niter = per_w // ch
    assert per_w % ch == 0 and niter >= 2

    xf = x.reshape(total)
    pf = pos.reshape(-1)[:pos_n]

    mesh = plsc.VectorSubcoreMesh(core_axis_name="c", subcore_axis_name="s")

    @functools.partial(
        pl.kernel,
        out_type=jax.ShapeDtypeStruct((total,), jnp.float32),
        mesh=mesh,
        scratch_types=[
            pltpu.VMEM((2, ch), jnp.float32),
            pltpu.VMEM((2, ch), jnp.float32),
            pltpu.SemaphoreType.DMA((2,)),
            pltpu.SemaphoreType.DMA((2,)),
            pltpu.SemaphoreType.DMA((2,)),
        ],
    )
    def k(x_hbm, p_hbm, o_hbm, xb, pb, xsem, psem, osem):
        wid = lax.axis_index("c") * ns + lax.axis_index("s")
        base = wid * per_w
        pbase = (wid % w_per_batch) * per_w

        def start_in(i, slot):
            off = base + i * ch
            poff = pbase + i * ch
            pltpu.async_copy(x_hbm.at[pl.ds(off, ch)], xb.at[slot], xsem.at[slot])
            pltpu.async_copy(p_hbm.at[pl.ds(poff, ch)], pb.at[slot], psem.at[slot])

        start_in(0, 0)

        @pl.loop(0, niter)
        def _(i):
            slot = lax.rem(i, 2)
            nslot = lax.rem(i + 1, 2)

            @pl.when(i + 1 < niter)
            def _():
                @pl.when(i >= 1)
                def _():
                    pltpu.make_async_copy(
                        xb.at[nslot],
                        o_hbm.at[pl.ds(base + (i - 1) * ch, ch)],
                        osem.at[nslot],
                    ).wait()

                start_in(i + 1, nslot)

            pltpu.make_async_copy(
                x_hbm.at[pl.ds(base + i * ch, ch)], xb.at[slot], xsem.at[slot]
            ).wait()
            pltpu.make_async_copy(
                p_hbm.at[pl.ds(pbase + i * ch, ch)], pb.at[slot], psem.at[slot]
            ).wait()

            @pl.loop(0, ch, step=16, unroll=8)
            def _(j):
                plsc.addupdate(xb.at[slot, pl.ds(j, 16)], pb[slot, pl.ds(j, 16)])

            pltpu.async_copy(
                xb.at[slot], o_hbm.at[pl.ds(base + i * ch, ch)], osem.at[slot]
            )

        pltpu.make_async_copy(
            xb.at[lax.rem(niter - 2, 2)],
            o_hbm.at[pl.ds(base + (niter - 2) * ch, ch)],
            osem.at[lax.rem(niter - 2, 2)],
        ).wait()
        pltpu.make_async_copy(
            xb.at[lax.rem(niter - 1, 2)],
            o_hbm.at[pl.ds(base + (niter - 1) * ch, ch)],
            osem.at[lax.rem(niter - 1, 2)],
        ).wait()

    return k(xf, pf).reshape(B, T, D)


# speedup vs baseline: 2.5315x; 2.1359x over previous
import functools
import jax
import jax.numpy as jnp
from jax import lax
from jax.experimental import pallas as pl
from jax.experimental.pallas import tpu as pltpu
from jax.experimental.pallas import tpu_sc as plsc

_TCH = 16


def kernel(x, pos):
    B, T, D = x.shape

    info = plsc.get_sparse_core_info()
    nc, ns = info.num_cores, info.num_subcores
    nw = nc * ns
    tpw = T // nw
    assert T % nw == 0
    tch = min(_TCH, tpw)
    ntc = tpw // tch
    assert tpw % tch == 0
    niter = ntc * B

    mesh = plsc.VectorSubcoreMesh(core_axis_name="c", subcore_axis_name="s")

    @functools.partial(
        pl.kernel,
        out_type=jax.ShapeDtypeStruct((B, T, D), jnp.float32),
        mesh=mesh,
        scratch_types=[
            pltpu.VMEM((2, tch, D), jnp.float32),
            pltpu.VMEM((2, tch, D), jnp.float32),
            pltpu.SemaphoreType.DMA((2,)),
            pltpu.SemaphoreType.DMA((2,)),
            pltpu.SemaphoreType.DMA((2,)),
        ],
    )
    def k(x_hbm, p_hbm, o_hbm, xb, pb, xsem, psem, osem):
        wid = lax.axis_index("c") * ns + lax.axis_index("s")
        t0w = wid * tpw

        def x_view(i):
            tc = lax.div(i, B)
            b = lax.rem(i, B)
            return x_hbm.at[b, pl.ds(t0w + tc * tch, tch), :]

        def o_view(i):
            tc = lax.div(i, B)
            b = lax.rem(i, B)
            return o_hbm.at[b, pl.ds(t0w + tc * tch, tch), :]

        def p_view(tc):
            return p_hbm.at[pl.ds(t0w + tc * tch, tch), :]

        def start_x(i, slot):
            pltpu.async_copy(x_view(i), xb.at[slot], xsem.at[slot])

        def start_p(tc, pslot):
            pltpu.async_copy(p_view(tc), pb.at[pslot], psem.at[pslot])

        start_p(0, 0)
        start_x(0, 0)

        @pl.loop(0, niter)
        def _(i):
            slot = lax.rem(i, 2)
            nslot = lax.rem(i + 1, 2)

            @pl.when(i + 1 < niter)
            def _():
                @pl.when(i >= 1)
                def _():
                    pltpu.make_async_copy(
                        xb.at[nslot], o_view(i - 1), osem.at[nslot]
                    ).wait()

                start_x(i + 1, nslot)

                @pl.when(lax.rem(i + 1, B) == 0)
                def _():
                    ntc_next = lax.div(i + 1, B)
                    start_p(ntc_next, lax.rem(ntc_next, 2))

            pltpu.make_async_copy(x_view(i), xb.at[slot], xsem.at[slot]).wait()

            tc = lax.div(i, B)
            pslot = lax.rem(tc, 2)

            @pl.when(lax.rem(i, B) == 0)
            def _():
                pltpu.make_async_copy(p_view(tc), pb.at[pslot], psem.at[pslot]).wait()

            @pl.loop(0, tch)
            def _(r):
                @pl.loop(0, D, step=16, unroll=8)
                def _(j):
                    plsc.addupdate(
                        xb.at[slot, r, pl.ds(j, 16)], pb[pslot, r, pl.ds(j, 16)]
                    )

            pltpu.async_copy(xb.at[slot], o_view(i), osem.at[slot])

        pltpu.make_async_copy(
            xb.at[lax.rem(niter - 2, 2)], o_view(niter - 2), osem.at[lax.rem(niter - 2, 2)]
        ).wait()
        pltpu.make_async_copy(
            xb.at[lax.rem(niter - 1, 2)], o_view(niter - 1), osem.at[lax.rem(niter - 1, 2)]
        ).wait()

    return k(x, pos[:T])


# speedup vs baseline: 8.4054x; 3.3203x over previous
import jax
import jax.numpy as jnp
from jax.experimental import pallas as pl


def _add_body(x_ref, pos_ref, o_ref):
    o_ref[...] = x_ref[...] + pos_ref[...]


def kernel(x, pos):
    B, T, D = x.shape
    TBLK = 2048
    nt = T // TBLK
    return pl.pallas_call(
        _add_body,
        grid=(nt, B),
        in_specs=[
            pl.BlockSpec((1, TBLK, D), lambda t, b: (b, t, 0)),
            pl.BlockSpec((TBLK, D), lambda t, b: (t, 0)),
        ],
        out_specs=pl.BlockSpec((1, TBLK, D), lambda t, b: (b, t, 0)),
        out_shape=jax.ShapeDtypeStruct(x.shape, x.dtype),
    )(x, pos)


# speedup vs baseline: 8.4573x; 1.0062x over previous
import jax
import jax.numpy as jnp
from jax.experimental import pallas as pl


def _add_body(x_ref, pos_ref, o_ref):
    o_ref[...] = x_ref[...] + pos_ref[...]


def kernel(x, pos):
    B, T, D = x.shape
    TBLK = 1024
    BBLK = 2
    nt = T // TBLK
    nb = B // BBLK
    return pl.pallas_call(
        _add_body,
        grid=(nt, nb),
        in_specs=[
            pl.BlockSpec((BBLK, TBLK, D), lambda t, b: (b, t, 0)),
            pl.BlockSpec((TBLK, D), lambda t, b: (t, 0)),
        ],
        out_specs=pl.BlockSpec((BBLK, TBLK, D), lambda t, b: (b, t, 0)),
        out_shape=jax.ShapeDtypeStruct(x.shape, x.dtype),
    )(x, pos)
